# baseline (speedup 1.0000x reference)
import functools

import jax
import jax.numpy as jnp
from jax import lax
from jax.experimental import pallas as pl
from jax.experimental.pallas import tpu as pltpu
from jax.experimental.pallas import tpu_sc as plsc

OUT_SIZE = 64
PAD_W = 128
BATCH = 16384
HIST = 50
HIST_P = 56

NC, NS = 2, 16
NW = NC * NS
B_PER_W = BATCH // NW
IDS_PER_W = B_PER_W * HIST_P
CB = 4
NCHUNK = B_PER_W // CB


def _gather_body(idx_hbm, table_hbm, out_hbm, idx_v, rows_v, sem):
    wid = lax.axis_index("s") * NC + lax.axis_index("c")
    pltpu.sync_copy(idx_hbm.at[wid], idx_v)
    b_base = wid * B_PER_W

    def body(j, carry):
        for i in range(CB):
            pltpu.async_copy(
                table_hbm.at[idx_v.at[pl.ds((j * CB + i) * HIST_P, HIST_P)]],
                rows_v.at[i], sem)
        for i in range(CB):
            pltpu.make_async_copy(
                table_hbm.at[idx_v.at[pl.ds((j * CB + i) * HIST_P, HIST_P)]],
                rows_v.at[i], sem).wait()
        pltpu.sync_copy(rows_v, out_hbm.at[pl.ds(b_base + j * CB, CB)])
        return carry

    lax.fori_loop(0, NCHUNK, body, 0)


@functools.partial(jax.jit, static_argnums=())
def _run(idx, table):
    k = pl.kernel(
        _gather_body,
        out_type=jax.ShapeDtypeStruct((BATCH, HIST_P, PAD_W), jnp.float32),
        mesh=plsc.VectorSubcoreMesh(core_axis_name="c", subcore_axis_name="s"),
        scratch_types=[
            pltpu.VMEM((IDS_PER_W,), jnp.int32),
            pltpu.VMEM((CB, HIST_P, PAD_W), jnp.float32),
            pltpu.SemaphoreType.DMA,
        ],
        compiler_params=pltpu.CompilerParams(use_tc_tiling_on_sc=False),
    )
    return k(idx, table)


def kernel(inputs, embeddings):
    idx = jnp.pad(inputs.astype(jnp.int32), ((0, 0), (0, HIST_P - HIST)))
    idx = idx.reshape(NW, IDS_PER_W)
    tbl128 = jnp.pad(embeddings, ((0, 0), (0, PAD_W - OUT_SIZE)))
    out = _run(idx, tbl128)
    return out[:, :HIST, :OUT_SIZE]


# speedup vs baseline: 1.0015x; 1.0015x over previous
import functools

import jax
import jax.numpy as jnp
from jax import lax
from jax.experimental import pallas as pl
from jax.experimental.pallas import tpu as pltpu
from jax.experimental.pallas import tpu_sc as plsc

OUT_SIZE = 64
PAD_W = 128
BATCH = 16384
HIST = 50
HIST_P = 56
ROWS = BATCH * HIST_P

NC, NS = 2, 16
NW = NC * NS
ROWS_W = ROWS // NW
STREAM = 128
NCHUNK = ROWS_W // STREAM
NBUF = 4


def _gather_body(idx_hbm, table_hbm, out_hbm, idx_v, rows_v, *sems):
    gsems, ssems = sems[:NBUF], sems[NBUF:]
    wid = lax.axis_index("s") * NC + lax.axis_index("c")
    pltpu.sync_copy(idx_hbm.at[wid], idx_v)
    out_base = wid * ROWS_W

    def issue_gather(j, b):
        pltpu.async_copy(table_hbm.at[idx_v.at[j]], rows_v.at[b], gsems[b])

    def wait_gather(j, b):
        pltpu.make_async_copy(
            table_hbm.at[idx_v.at[j]], rows_v.at[b], gsems[b]).wait()

    def store_descr(j, b):
        return (rows_v.at[b],
                out_hbm.at[pl.ds(out_base + j * STREAM, STREAM)], ssems[b])

    for b in range(NBUF - 1):
        issue_gather(b, b)

    def body(g, carry):
        for b in range(NBUF):
            j = g * NBUF + b
            bp = (b + NBUF - 1) % NBUF
            wait_gather(j, b)
            pltpu.async_copy(*store_descr(j, b))

            @pl.when(j >= 1)
            def _():
                pltpu.make_async_copy(*store_descr(j - 1, bp)).wait()

            @pl.when(j + NBUF - 1 < NCHUNK)
            def _():
                issue_gather(j + NBUF - 1, bp)
        return carry

    lax.fori_loop(0, NCHUNK // NBUF, body, 0)
    pltpu.make_async_copy(*store_descr(NCHUNK - 1, (NCHUNK - 1) % NBUF)).wait()


@functools.partial(jax.jit, static_argnums=())
def _run(idx, table):
    k = pl.kernel(
        _gather_body,
        out_type=jax.ShapeDtypeStruct((ROWS, PAD_W), jnp.float32),
        mesh=plsc.VectorSubcoreMesh(core_axis_name="c", subcore_axis_name="s"),
        scratch_types=[
            pltpu.VMEM((NCHUNK, STREAM), jnp.int32),
            pltpu.VMEM((NBUF, STREAM, PAD_W), jnp.float32),
        ] + [pltpu.SemaphoreType.DMA] * (2 * NBUF),
        compiler_params=pltpu.CompilerParams(use_tc_tiling_on_sc=False),
    )
    return k(idx, table)


def kernel(inputs, embeddings):
    idx = jnp.pad(inputs.astype(jnp.int32), ((0, 0), (0, HIST_P - HIST)))
    idx = idx.reshape(NW, NCHUNK, STREAM)
    tbl128 = jnp.pad(embeddings, ((0, 0), (0, PAD_W - OUT_SIZE)))
    out = _run(idx, tbl128)
    return out.reshape(BATCH, HIST_P, PAD_W)[:, :HIST, :OUT_SIZE]


# speedup vs baseline: 1.7610x; 1.7583x over previous
import functools

import jax
import jax.numpy as jnp
from jax import lax
from jax.experimental import pallas as pl
from jax.experimental.pallas import tpu as pltpu
from jax.experimental.pallas import tpu_sc as plsc

OUT_SIZE = 64
PAD_W = 128
BATCH = 16384
HIST = 50
HIST_P = 56
ROWS = BATCH * HIST_P

NC, NS = 2, 16
NW = NC * NS
ROWS_W = ROWS // NW
STREAM = 128
NCHUNK = ROWS_W // STREAM
NBUF = 4


def _gather_body(idx_hbm, table_hbm, out_hbm, idx_v, rows_v, *sems):
    gsems, ssems = sems[:NBUF], sems[NBUF:]
    wid = lax.axis_index("s") * NC + lax.axis_index("c")
    pltpu.sync_copy(idx_hbm.at[wid], idx_v)
    out_base = wid * ROWS_W

    def issue_gather(j, b):
        pltpu.async_copy(table_hbm.at[idx_v.at[j]], rows_v.at[b], gsems[b])

    def wait_gather(j, b):
        pltpu.make_async_copy(
            table_hbm.at[idx_v.at[j]], rows_v.at[b], gsems[b]).wait()

    def store_descr(j, b):
        return (rows_v.at[b],
                out_hbm.at[pl.ds(out_base + j * STREAM, STREAM), 0], ssems[b])

    for b in range(NBUF - 1):
        issue_gather(b, b)

    def body(g, carry):
        for b in range(NBUF):
            j = g * NBUF + b
            bp = (b + NBUF - 1) % NBUF
            wait_gather(j, b)
            pltpu.async_copy(*store_descr(j, b))

            @pl.when(j >= 1)
            def _():
                pltpu.make_async_copy(*store_descr(j - 1, bp)).wait()

            @pl.when(j + NBUF - 1 < NCHUNK)
            def _():
                issue_gather(j + NBUF - 1, bp)
        return carry

    lax.fori_loop(0, NCHUNK // NBUF, body, 0)
    pltpu.make_async_copy(*store_descr(NCHUNK - 1, (NCHUNK - 1) % NBUF)).wait()


@functools.partial(jax.jit, static_argnums=())
def _run(idx, table):
    k = pl.kernel(
        _gather_body,
        out_type=jax.ShapeDtypeStruct((ROWS, 2, OUT_SIZE), jnp.float32),
        mesh=plsc.VectorSubcoreMesh(core_axis_name="c", subcore_axis_name="s"),
        scratch_types=[
            pltpu.VMEM((NCHUNK, STREAM), jnp.int32),
            pltpu.VMEM((NBUF, STREAM, OUT_SIZE), jnp.float32),
        ] + [pltpu.SemaphoreType.DMA] * (2 * NBUF),
        compiler_params=pltpu.CompilerParams(use_tc_tiling_on_sc=False),
    )
    return k(idx, table)


def kernel(inputs, embeddings):
    idx = jnp.pad(inputs.astype(jnp.int32), ((0, 0), (0, HIST_P - HIST)))
    idx = (idx * 2).reshape(NW, NCHUNK, STREAM)
    tbl = jnp.pad(embeddings, ((0, 0), (0, PAD_W - OUT_SIZE)))
    tbl = tbl.reshape(2 * 1000000, OUT_SIZE)
    out = _run(idx, tbl)
    return out.reshape(BATCH, HIST_P, PAD_W)[:, :HIST, :OUT_SIZE]
